# baseline (speedup 1.0000x reference)
import functools

import jax
import jax.numpy as jnp
from jax import lax
from jax.experimental import pallas as pl
from jax.experimental.pallas import tpu as pltpu
from jax.experimental.pallas import tpu_sc as plsc

S_, H_ = 2048, 1024
NH, NKV, HD = 16, 4, 64
E_, K_, F_ = 64, 2, 1024
EPS = 1e-6
ALPHA = 1.702
LIMIT = 7.0
CAP = 160
GQ = NH // NKV
BR = 256
NSLOT = E_ * CAP
NEG = -1e30


def _qkv_body(x_ref, w1_ref, wq_ref, bq_ref, q_ref, k_ref, v_ref):
    x = x_ref[...]
    nrm = x * lax.rsqrt(jnp.mean(x * x, axis=-1, keepdims=True) + EPS) * w1_ref[...]
    qkv = jnp.dot(nrm, wq_ref[...], preferred_element_type=jnp.float32) + bq_ref[...]
    for h in range(NH):
        q_ref[h] = qkv[:, h * HD:(h + 1) * HD]
    for h in range(NKV):
        k_ref[h] = qkv[:, (NH + h) * HD:(NH + h + 1) * HD]
        v_ref[h] = qkv[:, (NH + NKV + h) * HD:(NH + NKV + h + 1) * HD]


def _qkv_call(hs, norm1_w, W_qkv, b_qkv):
    D = (NH + 2 * NKV) * HD
    return pl.pallas_call(
        _qkv_body,
        grid=(S_ // BR,),
        in_specs=[
            pl.BlockSpec((BR, H_), lambda i: (i, 0)),
            pl.BlockSpec((1, H_), lambda i: (0, 0)),
            pl.BlockSpec((H_, D), lambda i: (0, 0)),
            pl.BlockSpec((1, D), lambda i: (0, 0)),
        ],
        out_specs=(
            pl.BlockSpec((NH, BR, HD), lambda i: (0, i, 0)),
            pl.BlockSpec((NKV, BR, HD), lambda i: (0, i, 0)),
            pl.BlockSpec((NKV, BR, HD), lambda i: (0, i, 0)),
        ),
        out_shape=(
            jax.ShapeDtypeStruct((NH, S_, HD), jnp.float32),
            jax.ShapeDtypeStruct((NKV, S_, HD), jnp.float32),
            jax.ShapeDtypeStruct((NKV, S_, HD), jnp.float32),
        ),
    )(hs, norm1_w, W_qkv, b_qkv)


def _attn_body(sinks_ref, q_ref, k_ref, v_ref, cq_ref, sq_ref, ck_ref, sk_ref, o_ref):
    h = pl.program_id(0)
    qb = pl.program_id(1)
    q = q_ref[0]
    cq = cq_ref[...]
    sq = sq_ref[...]
    q1 = q[:, : HD // 2]
    q2 = q[:, HD // 2:]
    qr1 = q1 * cq - q2 * sq
    qr2 = q2 * cq + q1 * sq
    k = k_ref[0]
    ck = ck_ref[...]
    sk = sk_ref[...]
    k1 = k[:, : HD // 2]
    k2 = k[:, HD // 2:]
    kr1 = k1 * ck - k2 * sk
    kr2 = k2 * ck + k1 * sk
    dn = (((1,), (1,)), ((), ()))
    s = (lax.dot_general(qr1, kr1, dn, preferred_element_type=jnp.float32)
         + lax.dot_general(qr2, kr2, dn, preferred_element_type=jnp.float32)) * (HD ** -0.5)
    rows = qb * BR + lax.broadcasted_iota(jnp.int32, s.shape, 0)
    cols = lax.broadcasted_iota(jnp.int32, s.shape, 1)
    s = jnp.where(cols <= rows, s, NEG)
    sink = sinks_ref[h]
    m = jnp.maximum(jnp.max(s, axis=-1, keepdims=True), sink)
    p = jnp.exp(s - m)
    denom = jnp.sum(p, axis=-1, keepdims=True) + jnp.exp(sink - m)
    o_ref[0] = jnp.dot(p, v_ref[0], preferred_element_type=jnp.float32) / denom


def _attn_call(q3, k3, v3, cosf, sinf, sinks):
    return pl.pallas_call(
        _attn_body,
        grid=(NH, S_ // BR),
        in_specs=[
            pl.BlockSpec(memory_space=pltpu.SMEM),
            pl.BlockSpec((1, BR, HD), lambda h, qb: (h, qb, 0)),
            pl.BlockSpec((1, S_, HD), lambda h, qb: (h // GQ, 0, 0)),
            pl.BlockSpec((1, S_, HD), lambda h, qb: (h // GQ, 0, 0)),
            pl.BlockSpec((BR, HD // 2), lambda h, qb: (qb, 0)),
            pl.BlockSpec((BR, HD // 2), lambda h, qb: (qb, 0)),
            pl.BlockSpec((S_, HD // 2), lambda h, qb: (0, 0)),
            pl.BlockSpec((S_, HD // 2), lambda h, qb: (0, 0)),
        ],
        out_specs=pl.BlockSpec((1, BR, HD), lambda h, qb: (h, qb, 0)),
        out_shape=jax.ShapeDtypeStruct((NH, S_, HD), jnp.float32),
    )(sinks, q3, k3, v3, cosf, sinf, cosf, sinf)


def _oproj_body(attn_ref, wo_ref, bo_ref, res_ref, n2_ref, wr_ref, br_ref,
                r2_ref, nrm_ref, i1_ref, i2_ref, p1_ref, p2_ref):
    o = jnp.dot(attn_ref[0], wo_ref[0], preferred_element_type=jnp.float32)
    for h in range(1, NH):
        o = o + jnp.dot(attn_ref[h], wo_ref[h], preferred_element_type=jnp.float32)
    r2 = res_ref[...] + o + bo_ref[...]
    r2_ref[...] = r2
    nrm = r2 * lax.rsqrt(jnp.mean(r2 * r2, axis=-1, keepdims=True) + EPS) * n2_ref[...]
    nrm_ref[...] = nrm
    logits = jnp.dot(nrm, wr_ref[...], preferred_element_type=jnp.float32) + br_ref[...]
    ecols = lax.broadcasted_iota(jnp.int32, logits.shape, 1)
    m1 = jnp.max(logits, axis=-1, keepdims=True)
    i1 = jnp.min(jnp.where(logits >= m1, ecols, E_), axis=-1, keepdims=True)
    l2 = jnp.where(ecols == i1, NEG, logits)
    m2 = jnp.max(l2, axis=-1, keepdims=True)
    i2 = jnp.min(jnp.where(l2 >= m2, ecols, E_), axis=-1, keepdims=True)
    e = jnp.exp(m2 - m1)
    p1 = 1.0 / (1.0 + e)
    i1_ref[...] = i1
    i2_ref[...] = i2
    p1_ref[...] = p1
    p2_ref[...] = 1.0 - p1


def _oproj_call(attn, W_o, b_o, resid, norm2_w, W_r, b_r):
    return pl.pallas_call(
        _oproj_body,
        grid=(S_ // BR,),
        in_specs=[
            pl.BlockSpec((NH, BR, HD), lambda i: (0, i, 0)),
            pl.BlockSpec((NH, HD, H_), lambda i: (0, 0, 0)),
            pl.BlockSpec((1, H_), lambda i: (0, 0)),
            pl.BlockSpec((BR, H_), lambda i: (i, 0)),
            pl.BlockSpec((1, H_), lambda i: (0, 0)),
            pl.BlockSpec((H_, E_), lambda i: (0, 0)),
            pl.BlockSpec((1, E_), lambda i: (0, 0)),
        ],
        out_specs=(
            pl.BlockSpec((BR, H_), lambda i: (i, 0)),
            pl.BlockSpec((BR, H_), lambda i: (i, 0)),
            pl.BlockSpec((BR, 1), lambda i: (i, 0)),
            pl.BlockSpec((BR, 1), lambda i: (i, 0)),
            pl.BlockSpec((BR, 1), lambda i: (i, 0)),
            pl.BlockSpec((BR, 1), lambda i: (i, 0)),
        ),
        out_shape=(
            jax.ShapeDtypeStruct((S_, H_), jnp.float32),
            jax.ShapeDtypeStruct((S_, H_), jnp.float32),
            jax.ShapeDtypeStruct((S_, 1), jnp.int32),
            jax.ShapeDtypeStruct((S_, 1), jnp.int32),
            jax.ShapeDtypeStruct((S_, 1), jnp.float32),
            jax.ShapeDtypeStruct((S_, 1), jnp.float32),
        ),
    )(attn, W_o, b_o, resid, norm2_w, W_r, b_r)


@functools.cache
def _make_sc_gather(V, D, Bn, CH):
    NC, NS = 2, 16
    NW = NC * NS
    assert Bn % NW == 0
    b_per_w = Bn // NW
    assert b_per_w % CH == 0 and CH % 8 == 0
    mesh = plsc.VectorSubcoreMesh(core_axis_name="c", subcore_axis_name="s")

    n_ch = b_per_w // CH

    @functools.partial(
        pl.kernel, mesh=mesh,
        out_type=jax.ShapeDtypeStruct((Bn, D), jnp.float32),
        scratch_types=[
            pltpu.VMEM((b_per_w,), jnp.int32),
            pltpu.VMEM((CH, D), jnp.float32),
            pltpu.VMEM((CH, D), jnp.float32),
            pltpu.SemaphoreType.DMA,
            pltpu.SemaphoreType.DMA,
        ],
    )
    def k(table_hbm, idx_hbm, out_hbm, idx_v, rows_a, rows_b, sem_a, sem_b):
        wid = lax.axis_index("s") * NC + lax.axis_index("c")
        base = wid * b_per_w
        pltpu.sync_copy(idx_hbm.at[pl.ds(base, b_per_w)], idx_v)
        bufs = (rows_a, rows_b)
        sems = (sem_a, sem_b)
        copies = [None] * n_ch
        copies[0] = pltpu.async_copy(
            table_hbm.at[idx_v.at[pl.ds(0, CH)]], rows_a, sem_a)
        for c in range(n_ch):
            if c + 1 < n_ch:
                copies[c + 1] = pltpu.async_copy(
                    table_hbm.at[idx_v.at[pl.ds((c + 1) * CH, CH)]],
                    bufs[(c + 1) % 2], sems[(c + 1) % 2])
            copies[c].wait()
            pltpu.sync_copy(bufs[c % 2], out_hbm.at[pl.ds(base + c * CH, CH)])

    return k


def _sc_gather_dispatch(table, idx):
    return _make_sc_gather(S_, H_, NSLOT, 40)(table, idx)


def _sc_gather_combine(table, idx):
    return _make_sc_gather(NSLOT, H_, S_ * K_, 32)(table, idx)


def _ffn_body(x_ref, gu_ref, gub_ref, dn_ref, dnb_ref, y_ref):
    x = x_ref[0]
    mm_t = lax.dot_general(gu_ref[0], x, (((0,), (1,)), ((), ())),
                           preferred_element_type=jnp.float32) + gub_ref[0]
    mm3 = mm_t.reshape(F_, 2, CAP)
    gate = jnp.minimum(mm3[:, 0, :], LIMIT)
    up = jnp.clip(mm3[:, 1, :], -LIMIT, LIMIT)
    glu = gate * (1.0 / (1.0 + jnp.exp(-ALPHA * gate)))
    act_t = (up + 1.0) * glu
    y = lax.dot_general(act_t, dn_ref[0], (((0,), (0,)), ((), ())),
                        preferred_element_type=jnp.float32)
    y_ref[0] = y + dnb_ref[0]


def _ffn_call(xbuf, gate_up, gate_up_b, down, down_b):
    return pl.pallas_call(
        _ffn_body,
        grid=(E_,),
        in_specs=[
            pl.BlockSpec((1, CAP, H_), lambda e: (e, 0, 0)),
            pl.BlockSpec((1, H_, 2 * F_), lambda e: (e, 0, 0)),
            pl.BlockSpec((1, 2 * F_, 1), lambda e: (e, 0, 0)),
            pl.BlockSpec((1, F_, H_), lambda e: (e, 0, 0)),
            pl.BlockSpec((1, 1, H_), lambda e: (e, 0, 0)),
        ],
        out_specs=pl.BlockSpec((1, CAP, H_), lambda e: (e, 0, 0)),
        out_shape=jax.ShapeDtypeStruct((E_, CAP, H_), jnp.float32),
    )(xbuf, gate_up, gate_up_b, down, down_b)


def _comb_body(r2_ref, yp_ref, wa_ref, wb_ref, o_ref):
    yp = yp_ref[...].reshape(BR, K_, H_)
    o_ref[...] = r2_ref[...] + wa_ref[...] * yp[:, 0, :] + wb_ref[...] * yp[:, 1, :]


def _comb_call(r2, ypair, wa, wb):
    return pl.pallas_call(
        _comb_body,
        grid=(S_ // BR,),
        in_specs=[
            pl.BlockSpec((BR, H_), lambda i: (i, 0)),
            pl.BlockSpec((K_ * BR, H_), lambda i: (i, 0)),
            pl.BlockSpec((BR, 1), lambda i: (i, 0)),
            pl.BlockSpec((BR, 1), lambda i: (i, 0)),
        ],
        out_specs=pl.BlockSpec((BR, H_), lambda i: (i, 0)),
        out_shape=jax.ShapeDtypeStruct((S_, H_), jnp.float32),
    )(r2, ypair, wa, wb)


def kernel(hidden_states, cos, sin, norm1_w, W_qkv, b_qkv, sinks, W_o, b_o,
           norm2_w, W_r, b_r, gate_up, gate_up_b, down, down_b):
    hs = hidden_states.reshape(S_, H_)
    cosf = cos.reshape(S_, HD // 2)
    sinf = sin.reshape(S_, HD // 2)

    q3, k3, v3 = _qkv_call(hs, norm1_w.reshape(1, H_), W_qkv, b_qkv.reshape(1, -1))
    attn = _attn_call(q3, k3, v3, cosf, sinf, sinks)
    r2, nrm, i1, i2, p1, p2 = _oproj_call(
        attn, W_o.reshape(NH, HD, H_), b_o.reshape(1, H_), hs,
        norm2_w.reshape(1, H_), W_r, b_r.reshape(1, E_))

    T = S_
    ef = jnp.concatenate([i1, i2], axis=1).reshape(-1)
    wf = jnp.concatenate([p1, p2], axis=1).reshape(-1)
    tok = jnp.repeat(jnp.arange(T, dtype=jnp.int32), K_)
    oh = jax.nn.one_hot(ef, E_, dtype=jnp.int32)
    ppos = ((jnp.cumsum(oh, axis=0) - 1) * oh).sum(-1)
    valid = ppos < CAP
    slot = ef * CAP + jnp.where(valid, ppos, 0)
    src = (jnp.zeros((NSLOT + 1,), jnp.int32)
           .at[jnp.where(valid, slot, NSLOT)].set(tok))[:NSLOT]
    gs = jnp.where(valid, slot, 0)
    wv = jnp.where(valid, wf, 0.0).reshape(T, K_)

    xbuf = _sc_gather_dispatch(nrm, src).reshape(E_, CAP, H_)
    ybuf = _ffn_call(xbuf, gate_up, gate_up_b.reshape(E_, 2 * F_, 1),
                     down, down_b.reshape(E_, 1, H_)).reshape(NSLOT, H_)
    ypair = _sc_gather_combine(ybuf, gs)
    out = _comb_call(r2, ypair, wv[:, 0:1], wv[:, 1:2])
    return out.reshape(hidden_states.shape)


# speedup vs baseline: 1.2958x; 1.2958x over previous
import functools

import jax
import jax.numpy as jnp
from jax import lax
from jax.experimental import pallas as pl
from jax.experimental.pallas import tpu as pltpu
from jax.experimental.pallas import tpu_sc as plsc

S_, H_ = 2048, 1024
NH, NKV, HD = 16, 4, 64
E_, K_, F_ = 64, 2, 1024
EPS = 1e-6
ALPHA = 1.702
LIMIT = 7.0
CAP = 160
GQ = NH // NKV
BR = 256
NSLOT = E_ * CAP
NEG = -1e30


def _qkv_body(x_ref, w1_ref, wq_ref, bq_ref, q_ref, k_ref, v_ref):
    x = x_ref[...]
    nrm = x * lax.rsqrt(jnp.mean(x * x, axis=-1, keepdims=True) + EPS) * w1_ref[...]
    qkv = jnp.dot(nrm, wq_ref[...], preferred_element_type=jnp.float32) + bq_ref[...]
    for h in range(NH):
        q_ref[h] = qkv[:, h * HD:(h + 1) * HD]
    for h in range(NKV):
        k_ref[h] = qkv[:, (NH + h) * HD:(NH + h + 1) * HD]
        v_ref[h] = qkv[:, (NH + NKV + h) * HD:(NH + NKV + h + 1) * HD]


def _qkv_call(hs, norm1_w, W_qkv, b_qkv):
    D = (NH + 2 * NKV) * HD
    return pl.pallas_call(
        _qkv_body,
        grid=(S_ // BR,),
        in_specs=[
            pl.BlockSpec((BR, H_), lambda i: (i, 0)),
            pl.BlockSpec((1, H_), lambda i: (0, 0)),
            pl.BlockSpec((H_, D), lambda i: (0, 0)),
            pl.BlockSpec((1, D), lambda i: (0, 0)),
        ],
        out_specs=(
            pl.BlockSpec((NH, BR, HD), lambda i: (0, i, 0)),
            pl.BlockSpec((NKV, BR, HD), lambda i: (0, i, 0)),
            pl.BlockSpec((NKV, BR, HD), lambda i: (0, i, 0)),
        ),
        out_shape=(
            jax.ShapeDtypeStruct((NH, S_, HD), jnp.float32),
            jax.ShapeDtypeStruct((NKV, S_, HD), jnp.float32),
            jax.ShapeDtypeStruct((NKV, S_, HD), jnp.float32),
        ),
    )(hs, norm1_w, W_qkv, b_qkv)


def _attn_body(sinks_ref, q_ref, k_ref, v_ref, cq_ref, sq_ref, ck_ref, sk_ref, o_ref):
    h = pl.program_id(0)
    qb = pl.program_id(1)
    q = q_ref[0]
    cq = cq_ref[...]
    sq = sq_ref[...]
    q1 = q[:, : HD // 2]
    q2 = q[:, HD // 2:]
    qr1 = q1 * cq - q2 * sq
    qr2 = q2 * cq + q1 * sq
    k = k_ref[0]
    ck = ck_ref[...]
    sk = sk_ref[...]
    k1 = k[:, : HD // 2]
    k2 = k[:, HD // 2:]
    kr1 = k1 * ck - k2 * sk
    kr2 = k2 * ck + k1 * sk
    dn = (((1,), (1,)), ((), ()))
    s = (lax.dot_general(qr1, kr1, dn, preferred_element_type=jnp.float32)
         + lax.dot_general(qr2, kr2, dn, preferred_element_type=jnp.float32)) * (HD ** -0.5)
    rows = qb * BR + lax.broadcasted_iota(jnp.int32, s.shape, 0)
    cols = lax.broadcasted_iota(jnp.int32, s.shape, 1)
    s = jnp.where(cols <= rows, s, NEG)
    sink = sinks_ref[h]
    m = jnp.maximum(jnp.max(s, axis=-1, keepdims=True), sink)
    p = jnp.exp(s - m)
    denom = jnp.sum(p, axis=-1, keepdims=True) + jnp.exp(sink - m)
    o_ref[0] = jnp.dot(p, v_ref[0], preferred_element_type=jnp.float32) / denom


def _attn_call(q3, k3, v3, cosf, sinf, sinks):
    return pl.pallas_call(
        _attn_body,
        grid=(NH, S_ // BR),
        in_specs=[
            pl.BlockSpec(memory_space=pltpu.SMEM),
            pl.BlockSpec((1, BR, HD), lambda h, qb: (h, qb, 0)),
            pl.BlockSpec((1, S_, HD), lambda h, qb: (h // GQ, 0, 0)),
            pl.BlockSpec((1, S_, HD), lambda h, qb: (h // GQ, 0, 0)),
            pl.BlockSpec((BR, HD // 2), lambda h, qb: (qb, 0)),
            pl.BlockSpec((BR, HD // 2), lambda h, qb: (qb, 0)),
            pl.BlockSpec((S_, HD // 2), lambda h, qb: (0, 0)),
            pl.BlockSpec((S_, HD // 2), lambda h, qb: (0, 0)),
        ],
        out_specs=pl.BlockSpec((1, BR, HD), lambda h, qb: (h, qb, 0)),
        out_shape=jax.ShapeDtypeStruct((NH, S_, HD), jnp.float32),
    )(sinks, q3, k3, v3, cosf, sinf, cosf, sinf)


def _oproj_body(attn_ref, wo_ref, bo_ref, res_ref, n2_ref, wr_ref, br_ref,
                r2_ref, nrm_ref, i1_ref, i2_ref, p1_ref, p2_ref):
    o = jnp.dot(attn_ref[0], wo_ref[0], preferred_element_type=jnp.float32)
    for h in range(1, NH):
        o = o + jnp.dot(attn_ref[h], wo_ref[h], preferred_element_type=jnp.float32)
    r2 = res_ref[...] + o + bo_ref[...]
    r2_ref[...] = r2
    nrm = r2 * lax.rsqrt(jnp.mean(r2 * r2, axis=-1, keepdims=True) + EPS) * n2_ref[...]
    nrm_ref[...] = nrm
    logits = jnp.dot(nrm, wr_ref[...], preferred_element_type=jnp.float32) + br_ref[...]
    ecols = lax.broadcasted_iota(jnp.int32, logits.shape, 1)
    m1 = jnp.max(logits, axis=-1, keepdims=True)
    i1 = jnp.min(jnp.where(logits >= m1, ecols, E_), axis=-1, keepdims=True)
    l2 = jnp.where(ecols == i1, NEG, logits)
    m2 = jnp.max(l2, axis=-1, keepdims=True)
    i2 = jnp.min(jnp.where(l2 >= m2, ecols, E_), axis=-1, keepdims=True)
    e = jnp.exp(m2 - m1)
    p1 = 1.0 / (1.0 + e)
    i1_ref[...] = i1
    i2_ref[...] = i2
    p1_ref[...] = p1
    p2_ref[...] = 1.0 - p1


def _oproj_call(attn, W_o, b_o, resid, norm2_w, W_r, b_r):
    return pl.pallas_call(
        _oproj_body,
        grid=(S_ // BR,),
        in_specs=[
            pl.BlockSpec((NH, BR, HD), lambda i: (0, i, 0)),
            pl.BlockSpec((NH, HD, H_), lambda i: (0, 0, 0)),
            pl.BlockSpec((1, H_), lambda i: (0, 0)),
            pl.BlockSpec((BR, H_), lambda i: (i, 0)),
            pl.BlockSpec((1, H_), lambda i: (0, 0)),
            pl.BlockSpec((H_, E_), lambda i: (0, 0)),
            pl.BlockSpec((1, E_), lambda i: (0, 0)),
        ],
        out_specs=(
            pl.BlockSpec((BR, H_), lambda i: (i, 0)),
            pl.BlockSpec((BR, H_), lambda i: (i, 0)),
            pl.BlockSpec((BR, 1), lambda i: (i, 0)),
            pl.BlockSpec((BR, 1), lambda i: (i, 0)),
            pl.BlockSpec((BR, 1), lambda i: (i, 0)),
            pl.BlockSpec((BR, 1), lambda i: (i, 0)),
        ),
        out_shape=(
            jax.ShapeDtypeStruct((S_, H_), jnp.float32),
            jax.ShapeDtypeStruct((S_, H_), jnp.float32),
            jax.ShapeDtypeStruct((S_, 1), jnp.int32),
            jax.ShapeDtypeStruct((S_, 1), jnp.int32),
            jax.ShapeDtypeStruct((S_, 1), jnp.float32),
            jax.ShapeDtypeStruct((S_, 1), jnp.float32),
        ),
    )(attn, W_o, b_o, resid, norm2_w, W_r, b_r)


@functools.cache
def _make_sc_gather(V, D, Bn, CH):
    NC, NS = 2, 16
    NW = NC * NS
    assert Bn % NW == 0
    b_per_w = Bn // NW
    assert b_per_w % CH == 0 and CH % 8 == 0
    mesh = plsc.VectorSubcoreMesh(core_axis_name="c", subcore_axis_name="s")

    n_ch = b_per_w // CH

    @functools.partial(
        pl.kernel, mesh=mesh,
        out_type=jax.ShapeDtypeStruct((Bn, D), jnp.float32),
        scratch_types=[
            pltpu.VMEM((b_per_w,), jnp.int32),
            pltpu.VMEM((CH, D), jnp.float32),
            pltpu.VMEM((CH, D), jnp.float32),
            pltpu.SemaphoreType.DMA,
            pltpu.SemaphoreType.DMA,
        ],
    )
    def k(table_hbm, idx_hbm, out_hbm, idx_v, rows_a, rows_b, sem_a, sem_b):
        wid = lax.axis_index("s") * NC + lax.axis_index("c")
        base = wid * b_per_w
        pltpu.sync_copy(idx_hbm.at[pl.ds(base, b_per_w)], idx_v)
        bufs = (rows_a, rows_b)
        sems = (sem_a, sem_b)
        copies = [None] * n_ch
        copies[0] = pltpu.async_copy(
            table_hbm.at[idx_v.at[pl.ds(0, CH)]], rows_a, sem_a)
        for c in range(n_ch):
            if c + 1 < n_ch:
                copies[c + 1] = pltpu.async_copy(
                    table_hbm.at[idx_v.at[pl.ds((c + 1) * CH, CH)]],
                    bufs[(c + 1) % 2], sems[(c + 1) % 2])
            copies[c].wait()
            pltpu.sync_copy(bufs[c % 2], out_hbm.at[pl.ds(base + c * CH, CH)])

    return k


def _sc_gather_dispatch(table, idx):
    return _make_sc_gather(S_, H_, NSLOT, 40)(table, idx)


def _sc_gather_combine(table, idx):
    return _make_sc_gather(NSLOT, H_, S_ * K_, 32)(table, idx)


def _ffn_body(x_ref, gu_ref, gub_ref, dn_ref, dnb_ref, y_ref):
    x = x_ref[0]
    mm_t = lax.dot_general(gu_ref[0], x, (((0,), (1,)), ((), ())),
                           preferred_element_type=jnp.float32) + gub_ref[0]
    mm3 = mm_t.reshape(F_, 2, CAP)
    gate = jnp.minimum(mm3[:, 0, :], LIMIT)
    up = jnp.clip(mm3[:, 1, :], -LIMIT, LIMIT)
    glu = gate * (1.0 / (1.0 + jnp.exp(-ALPHA * gate)))
    act_t = (up + 1.0) * glu
    y = lax.dot_general(act_t, dn_ref[0], (((0,), (0,)), ((), ())),
                        preferred_element_type=jnp.float32)
    y_ref[0] = y + dnb_ref[0]


def _ffn_call(xbuf, gate_up, gate_up_b, down, down_b):
    return pl.pallas_call(
        _ffn_body,
        grid=(E_,),
        in_specs=[
            pl.BlockSpec((1, CAP, H_), lambda e: (e, 0, 0)),
            pl.BlockSpec((1, H_, 2 * F_), lambda e: (e, 0, 0)),
            pl.BlockSpec((1, 2 * F_, 1), lambda e: (e, 0, 0)),
            pl.BlockSpec((1, F_, H_), lambda e: (e, 0, 0)),
            pl.BlockSpec((1, 1, H_), lambda e: (e, 0, 0)),
        ],
        out_specs=pl.BlockSpec((1, CAP, H_), lambda e: (e, 0, 0)),
        out_shape=jax.ShapeDtypeStruct((E_, CAP, H_), jnp.float32),
    )(xbuf, gate_up, gate_up_b, down, down_b)


def _comb_body(r2_ref, yp_ref, wa_ref, wb_ref, o_ref):
    yp = yp_ref[...].reshape(BR, K_, H_)
    o_ref[...] = r2_ref[...] + wa_ref[...] * yp[:, 0, :] + wb_ref[...] * yp[:, 1, :]


def _comb_call(r2, ypair, wa, wb):
    return pl.pallas_call(
        _comb_body,
        grid=(S_ // BR,),
        in_specs=[
            pl.BlockSpec((BR, H_), lambda i: (i, 0)),
            pl.BlockSpec((K_ * BR, H_), lambda i: (i, 0)),
            pl.BlockSpec((BR, 1), lambda i: (i, 0)),
            pl.BlockSpec((BR, 1), lambda i: (i, 0)),
        ],
        out_specs=pl.BlockSpec((BR, H_), lambda i: (i, 0)),
        out_shape=jax.ShapeDtypeStruct((S_, H_), jnp.float32),
    )(r2, ypair, wa, wb)


def kernel(hidden_states, cos, sin, norm1_w, W_qkv, b_qkv, sinks, W_o, b_o,
           norm2_w, W_r, b_r, gate_up, gate_up_b, down, down_b):
    hs = hidden_states.reshape(S_, H_)
    cosf = cos.reshape(S_, HD // 2)
    sinf = sin.reshape(S_, HD // 2)

    q3, k3, v3 = _qkv_call(hs, norm1_w.reshape(1, H_), W_qkv, b_qkv.reshape(1, -1))
    attn = _attn_call(q3, k3, v3, cosf, sinf, sinks)
    r2, nrm, i1, i2, p1, p2 = _oproj_call(
        attn, W_o.reshape(NH, HD, H_), b_o.reshape(1, H_), hs,
        norm2_w.reshape(1, H_), W_r, b_r.reshape(1, E_))

    T = S_
    ef = jnp.concatenate([i1, i2], axis=1).reshape(-1)
    wf = jnp.concatenate([p1, p2], axis=1).reshape(-1)
    tok = jnp.repeat(jnp.arange(T, dtype=jnp.int32), K_)
    oh = jax.nn.one_hot(ef, E_, dtype=jnp.int32)
    ppos = ((jnp.cumsum(oh, axis=0) - 1) * oh).sum(-1)
    valid = ppos < CAP
    slot = ef * CAP + jnp.where(valid, ppos, 0)
    src = ((jnp.arange(NSLOT + 1, dtype=jnp.int32) % S_)
           .at[jnp.where(valid, slot, NSLOT)].set(tok))[:NSLOT]
    gs = jnp.where(valid, slot, 0)
    wv = jnp.where(valid, wf, 0.0).reshape(T, K_)

    xbuf = _sc_gather_dispatch(nrm, src).reshape(E_, CAP, H_)
    ybuf = _ffn_call(xbuf, gate_up, gate_up_b.reshape(E_, 2 * F_, 1),
                     down, down_b.reshape(E_, 1, H_)).reshape(NSLOT, H_)
    ypair = _sc_gather_combine(ybuf, gs)
    out = _comb_call(r2, ypair, wv[:, 0:1], wv[:, 1:2])
    return out.reshape(hidden_states.shape)
